# baseline (speedup 1.0000x reference)
import functools

import jax
import jax.numpy as jnp
from jax.experimental import pallas as pl
from jax.experimental.pallas import tpu as pltpu

_CUTOFF = 5.0
_TI = 32


def _silu(x):
    return x / (1.0 + jnp.exp(-x))


def _newton_kernel(
    d_ref, mask_ref, dvt_ref, node0_ref, wie_ref, wbig_ref, small_ref,
    node_out, eqF_out, eqf_out, eqdr_out,
    node_old_s, imn_s, node_new_s, eqdr_old_s, eqdr_new_s, eqf_s, eqF_s,
    *, A, F, NB, TI,
):
    l = pl.program_id(0)
    b = pl.program_id(1)
    ib = pl.program_id(2)
    e = TI * A

    def w(k):
        return wbig_ref[0, k]

    def sv(k):
        return small_ref[0, k:k + 1, :]

    @pl.when(ib == 0)
    def _prologue():
        @pl.when(l == 0)
        def _():
            node_old_s[...] = node0_ref[0]
            eqdr_old_s[...] = jnp.zeros_like(eqdr_old_s)
            eqf_s[b] = jnp.zeros_like(eqf_s[0])
            eqF_s[b] = jnp.zeros_like(eqF_s[0])

        @pl.when(l > 0)
        def _():
            node_old_s[...] = node_new_s[b]
            eqdr_old_s[...] = eqdr_new_s[b]

        n_old = node_old_s[...]
        h = _silu(jnp.dot(n_old, w(0), preferred_element_type=jnp.float32) + sv(1))
        imn_s[...] = jnp.dot(h, w(1), preferred_element_type=jnp.float32) + sv(2)

    rows = pl.ds(ib * TI, TI)

    d2 = d_ref[0]
    mask2 = mask_ref[0]
    dvt = dvt_ref[0]

    x = d2 * (1.0 / _CUTOFF)
    x2 = x * x
    x3 = x2 * x
    x6 = x3 * x3
    cut = 1.0 - 28.0 * x6 + 48.0 * x6 * x - 21.0 * x6 * x2
    cm = jnp.where(d2 < _CUTOFF, cut, 0.0) * mask2

    dcol = d2[..., None]
    nvec = jax.lax.broadcasted_iota(
        jnp.int32, (1, 1, NB), 2).astype(jnp.float32) + 1.0
    S3 = jnp.sin(dcol * (jnp.pi / _CUTOFF) * nvec) * (
        (2.0 / _CUTOFF) ** 0.5) / (dcol + 1e-8)
    S = S3.reshape(e, NB)
    ime = jnp.dot(S, wie_ref[0], preferred_element_type=jnp.float32) + sv(0)

    imn_i = imn_s[rows, :]
    imn_all = imn_s[...]
    msg3 = (ime.reshape(TI, A, F) * cm[..., None]
            * imn_i[:, None, :] * imn_all[None, :, :])
    msg = msg3.reshape(e, F)

    node1 = node_old_s[rows, :] + jnp.sum(msg3, axis=1)

    s2 = jnp.sum(msg3 * sv(11)[None], axis=2)
    emF = dvt * s2[None]

    eqF_blk = eqF_s[b, rows, :] + jnp.concatenate(
        [jnp.sum(emF[c], axis=1, keepdims=True) for c in range(3)], axis=1)
    eqF_s[b, rows, :] = eqF_blk

    h1 = _silu(jnp.dot(msg, w(2), preferred_element_type=jnp.float32) + sv(3))
    feat3 = (jnp.dot(h1, w(3), preferred_element_type=jnp.float32)
             + sv(4)).reshape(TI, A, F)
    updf = [jnp.sum(feat3 * emF[c][..., None], axis=1) for c in range(3)]

    h2 = _silu(jnp.dot(msg, w(8), preferred_element_type=jnp.float32))
    edr3 = jnp.dot(h2, w(9), preferred_element_type=jnp.float32).reshape(TI, A, F)
    upddr = [jnp.sum(edr3 * eqdr_old_s[c][None], axis=1) for c in range(3)]

    hsu = _silu(jnp.dot(node1, w(4), preferred_element_type=jnp.float32) + sv(5))
    su = jnp.dot(hsu, w(5), preferred_element_type=jnp.float32) + sv(6)
    his = _silu(jnp.dot(node1, w(6), preferred_element_type=jnp.float32) + sv(7))
    isc = jnp.dot(his, w(7), preferred_element_type=jnp.float32) + sv(8)

    dot = jnp.zeros((TI, F), dtype=jnp.float32)
    for c in range(3):
        eqf_c = eqf_s[b, c, rows, :] + updf[c]
        eqf_s[b, c, rows, :] = eqf_c
        eqdr_c = eqdr_old_s[c, rows, :] + upddr[c] + su * updf[c]
        eqdr_new_s[b, c, rows, :] = eqdr_c
        eqf_out[0, c] = eqf_c
        eqdr_out[0, c] = eqdr_c
        dot = dot + eqf_c * eqdr_c

    node2 = node1 - isc * dot
    mu = jnp.mean(node2, axis=-1, keepdims=True)
    xc = node2 - mu
    var = jnp.mean(xc * xc, axis=-1, keepdims=True)
    node3 = sv(9) * xc * jax.lax.rsqrt(var + 1e-5) + sv(10)

    node_new_s[b, rows, :] = node3
    node_out[0] = node3
    eqF_out[0] = eqF_blk


def kernel(atomic_numbers, positions, neighbor_mask, distances,
           distance_vectors, params):
    del positions
    layers = params['layers']
    NL = len(layers)
    B, A = distances.shape[:2]
    F = params['emb'].shape[1]
    NB = layers[0]['Wie'].shape[0]
    TI = _TI
    NI = A // TI

    node0 = jnp.take(params['emb'], atomic_numbers, axis=0)
    dvt = jnp.transpose(
        distance_vectors / (distances[..., None] + 1e-8), (0, 3, 1, 2))

    wie = jnp.stack([lp['Wie'] for lp in layers])
    wbig = jnp.stack([
        jnp.stack([lp['mn_W1'], lp['mn_W2'], lp['mf_W1'], lp['mf_W2'],
                   lp['su_W1'], lp['su_W2'], lp['is_W1'], lp['is_W2'],
                   lp['me_W1'], lp['me_W2']])
        for lp in layers])
    zrow = jnp.zeros((F,), jnp.float32)
    small = jnp.stack([
        jnp.stack([lp['bie'], lp['mn_b1'], lp['mn_b2'], lp['mf_b1'],
                   lp['mf_b2'], lp['su_b1'], lp['su_b2'], lp['is_b1'],
                   lp['is_b2'], lp['ln_g'], lp['ln_b'], lp['Wc'][:, 0],
                   zrow, zrow, zrow, zrow])
        for lp in layers])

    f32 = jnp.float32
    grid = (NL, B, NI)
    out_shapes = (
        jax.ShapeDtypeStruct((B, A, F), f32),
        jax.ShapeDtypeStruct((B, A, 3), f32),
        jax.ShapeDtypeStruct((B, 3, A, F), f32),
        jax.ShapeDtypeStruct((B, 3, A, F), f32),
    )
    in_specs = [
        pl.BlockSpec((1, TI, A), lambda l, b, ib: (b, ib, 0)),
        pl.BlockSpec((1, TI, A), lambda l, b, ib: (b, ib, 0)),
        pl.BlockSpec((1, 3, TI, A), lambda l, b, ib: (b, 0, ib, 0)),
        pl.BlockSpec((1, A, F), lambda l, b, ib: (b, 0, 0)),
        pl.BlockSpec((1, NB, F), lambda l, b, ib: (l, 0, 0)),
        pl.BlockSpec((1, 10, F, F), lambda l, b, ib: (l, 0, 0, 0)),
        pl.BlockSpec((1, 16, F), lambda l, b, ib: (l, 0, 0)),
    ]
    out_specs = (
        pl.BlockSpec((1, TI, F), lambda l, b, ib: (b, ib, 0)),
        pl.BlockSpec((1, TI, 3), lambda l, b, ib: (b, ib, 0)),
        pl.BlockSpec((1, 3, TI, F), lambda l, b, ib: (b, 0, ib, 0)),
        pl.BlockSpec((1, 3, TI, F), lambda l, b, ib: (b, 0, ib, 0)),
    )
    scratch = [
        pltpu.VMEM((A, F), f32),
        pltpu.VMEM((A, F), f32),
        pltpu.VMEM((B, A, F), f32),
        pltpu.VMEM((3, A, F), f32),
        pltpu.VMEM((B, 3, A, F), f32),
        pltpu.VMEM((B, 3, A, F), f32),
        pltpu.VMEM((B, A, 3), f32),
    ]

    node, eqF, eqf_t, eqdr_t = pl.pallas_call(
        functools.partial(_newton_kernel, A=A, F=F, NB=NB, TI=TI),
        grid=grid,
        in_specs=in_specs,
        out_specs=out_specs,
        out_shape=out_shapes,
        scratch_shapes=scratch,
    )(distances, neighbor_mask, dvt, node0, wie, wbig, small)

    eqf = jnp.transpose(eqf_t, (0, 2, 1, 3))
    eqdr = jnp.transpose(eqdr_t, (0, 2, 1, 3))
    return node, eqF, eqf, eqdr


# speedup vs baseline: 2.5514x; 2.5514x over previous
import functools

import jax
import jax.numpy as jnp
from jax.experimental import pallas as pl
from jax.experimental.pallas import tpu as pltpu

_CUTOFF = 5.0
_TI = 32


def _silu(x):
    return x / (1.0 + jnp.exp(-x))


def _newton_kernel(
    dsq_ref, msq_ref, dvt_ref, node0_ref, wie_ref, wbig_ref,
    small_ref,
    node_out, eqF_out, eqf_out, eqdr_out,
    node_old_s, imn_s, node_new_s, eqdr_old_s, eqdr_new_s, eqf_s, eqF_s,
    *, A, F, NB, TI,
):
    b = pl.program_id(0)
    l = pl.program_id(1)
    ib = pl.program_id(2)
    e = TI * A

    def w(k):
        return wbig_ref[0, k]

    def sv(k):
        return small_ref[0, k:k + 1, :]

    @pl.when(ib == 0)
    def _prologue():
        @pl.when(l == 0)
        def _():
            node_old_s[...] = node0_ref[0]
            eqdr_old_s[...] = jnp.zeros_like(eqdr_old_s)
            eqf_s[b] = jnp.zeros_like(eqf_s[0])
            eqF_s[b] = jnp.zeros_like(eqF_s[0])

        @pl.when(l > 0)
        def _():
            node_old_s[...] = node_new_s[b]
            eqdr_old_s[...] = eqdr_new_s[b]

        n_old = node_old_s[...]
        h = _silu(jnp.dot(n_old, w(0), preferred_element_type=jnp.float32) + sv(1))
        imn_s[...] = jnp.dot(h, w(1), preferred_element_type=jnp.float32) + sv(2)

    rows = pl.ds(ib * TI, TI)

    dvt = dvt_ref[0]

    dsq = dsq_ref[0, 0]
    x = dsq * (1.0 / _CUTOFF)
    x2 = x * x
    x6 = x2 * x2 * x2
    cut = 1.0 - 28.0 * x6 + 48.0 * x6 * x - 21.0 * x6 * x2
    cm = jnp.where(dsq < _CUTOFF, cut, 0.0) * msq_ref[0, 0]
    th = dsq * (jnp.pi / _CUTOFF)
    s_a = jnp.sin(th)
    c2 = 2.0 * jnp.cos(th)
    rc = ((2.0 / _CUTOFF) ** 0.5) / (dsq + 1e-8) * cm
    basis = [s_a * rc]
    s_b = c2 * s_a
    basis.append(s_b * rc)
    for _ in range(NB - 2):
        s_a, s_b = s_b, c2 * s_b - s_a
        basis.append(s_b * rc)
    basis.append(cm)
    for _ in range(24 - NB - 1):
        basis.append(jnp.zeros_like(dsq))
    T = jnp.stack(basis, axis=0).reshape(24, e)
    ime = jax.lax.dot_general(
        T, wie_ref[0], (((0,), (0,)), ((), ())),
        preferred_element_type=jnp.float32)

    imn_i = imn_s[rows, :]
    imn_all = imn_s[...]
    msg3 = (ime.reshape(TI, A, F)
            * imn_i[:, None, :] * imn_all[None, :, :])
    msg = msg3.reshape(e, F)

    node1 = node_old_s[rows, :] + jnp.sum(msg3, axis=1)

    s2 = jnp.sum(msg3 * sv(11)[None], axis=2)
    emF = dvt * s2[None]

    eqF_blk = eqF_s[b, rows, :] + jnp.concatenate(
        [jnp.sum(emF[c], axis=1, keepdims=True) for c in range(3)], axis=1)
    eqF_s[b, rows, :] = eqF_blk

    h1 = _silu(jnp.dot(msg, w(2), preferred_element_type=jnp.float32) + sv(3))
    feat3 = (jnp.dot(h1, w(3), preferred_element_type=jnp.float32)
             + sv(4)).reshape(TI, A, F)
    updf = [jnp.sum(feat3 * emF[c][..., None], axis=1) for c in range(3)]

    def _with_me():
        h2 = _silu(jnp.dot(msg, w(8), preferred_element_type=jnp.float32))
        edr3 = jnp.dot(
            h2, w(9), preferred_element_type=jnp.float32).reshape(TI, A, F)
        return tuple(
            jnp.sum(edr3 * eqdr_old_s[c][None], axis=1) for c in range(3))

    def _no_me():
        z = jnp.zeros((TI, F), dtype=jnp.float32)
        return (z, z, z)

    upddr = jax.lax.cond(l > 0, _with_me, _no_me)

    hsu = _silu(jnp.dot(node1, w(4), preferred_element_type=jnp.float32) + sv(5))
    su = jnp.dot(hsu, w(5), preferred_element_type=jnp.float32) + sv(6)
    his = _silu(jnp.dot(node1, w(6), preferred_element_type=jnp.float32) + sv(7))
    isc = jnp.dot(his, w(7), preferred_element_type=jnp.float32) + sv(8)

    dot = jnp.zeros((TI, F), dtype=jnp.float32)
    for c in range(3):
        eqf_c = eqf_s[b, c, rows, :] + updf[c]
        eqf_s[b, c, rows, :] = eqf_c
        eqdr_c = eqdr_old_s[c, rows, :] + upddr[c] + su * updf[c]
        eqdr_new_s[b, c, rows, :] = eqdr_c
        eqf_out[0, c] = eqf_c
        eqdr_out[0, c] = eqdr_c
        dot = dot + eqf_c * eqdr_c

    node2 = node1 - isc * dot
    mu = jnp.mean(node2, axis=-1, keepdims=True)
    xc = node2 - mu
    var = jnp.mean(xc * xc, axis=-1, keepdims=True)
    node3 = sv(9) * xc * jax.lax.rsqrt(var + 1e-5) + sv(10)

    node_new_s[b, rows, :] = node3
    node_out[0] = node3
    eqF_out[0] = eqF_blk


def kernel(atomic_numbers, positions, neighbor_mask, distances,
           distance_vectors, params):
    del positions
    layers = params['layers']
    NL = len(layers)
    B, A = distances.shape[:2]
    F = params['emb'].shape[1]
    NB = layers[0]['Wie'].shape[0]
    TI = _TI
    NI = A // TI

    node0 = jnp.take(params['emb'], atomic_numbers, axis=0)
    dvt = jnp.transpose(
        distance_vectors / (distances[..., None] + 1e-8), (0, 3, 1, 2))
    ESUB = TI * A // 128
    dsq = distances.reshape(B, NI, ESUB, 128)
    msq = neighbor_mask.reshape(B, NI, ESUB, 128)

    wie = jnp.stack([
        jnp.concatenate([lp['Wie'], lp['bie'][None, :],
                         jnp.zeros((24 - NB - 1, F), jnp.float32)])
        for lp in layers])
    wbig = jnp.stack([
        jnp.stack([lp['mn_W1'], lp['mn_W2'], lp['mf_W1'], lp['mf_W2'],
                   lp['su_W1'], lp['su_W2'], lp['is_W1'], lp['is_W2'],
                   lp['me_W1'], lp['me_W2']])
        for lp in layers])
    zrow = jnp.zeros((F,), jnp.float32)
    small = jnp.stack([
        jnp.stack([lp['bie'], lp['mn_b1'], lp['mn_b2'], lp['mf_b1'],
                   lp['mf_b2'], lp['su_b1'], lp['su_b2'], lp['is_b1'],
                   lp['is_b2'], lp['ln_g'], lp['ln_b'], lp['Wc'][:, 0],
                   zrow, zrow, zrow, zrow])
        for lp in layers])

    f32 = jnp.float32
    grid = (B, NL, NI)
    out_shapes = (
        jax.ShapeDtypeStruct((B, A, F), f32),
        jax.ShapeDtypeStruct((B, A, 3), f32),
        jax.ShapeDtypeStruct((B, 3, A, F), f32),
        jax.ShapeDtypeStruct((B, 3, A, F), f32),
    )
    in_specs = [
        pl.BlockSpec((1, 1, ESUB, 128), lambda b, l, ib: (b, ib, 0, 0)),
        pl.BlockSpec((1, 1, ESUB, 128), lambda b, l, ib: (b, ib, 0, 0)),
        pl.BlockSpec((1, 3, TI, A), lambda b, l, ib: (b, 0, ib, 0)),
        pl.BlockSpec((1, A, F), lambda b, l, ib: (b, 0, 0)),
        pl.BlockSpec((1, 24, F), lambda b, l, ib: (l, 0, 0)),
        pl.BlockSpec((1, 10, F, F), lambda b, l, ib: (l, 0, 0, 0)),
        pl.BlockSpec((1, 16, F), lambda b, l, ib: (l, 0, 0)),
    ]
    out_specs = (
        pl.BlockSpec((1, TI, F), lambda b, l, ib: (b, ib, 0)),
        pl.BlockSpec((1, TI, 3), lambda b, l, ib: (b, ib, 0)),
        pl.BlockSpec((1, 3, TI, F), lambda b, l, ib: (b, 0, ib, 0)),
        pl.BlockSpec((1, 3, TI, F), lambda b, l, ib: (b, 0, ib, 0)),
    )
    scratch = [
        pltpu.VMEM((A, F), f32),
        pltpu.VMEM((A, F), f32),
        pltpu.VMEM((B, A, F), f32),
        pltpu.VMEM((3, A, F), f32),
        pltpu.VMEM((B, 3, A, F), f32),
        pltpu.VMEM((B, 3, A, F), f32),
        pltpu.VMEM((B, A, 3), f32),
    ]

    node, eqF, eqf_t, eqdr_t = pl.pallas_call(
        functools.partial(_newton_kernel, A=A, F=F, NB=NB, TI=TI),
        grid=grid,
        in_specs=in_specs,
        out_specs=out_specs,
        out_shape=out_shapes,
        scratch_shapes=scratch,
        compiler_params=pltpu.CompilerParams(
            dimension_semantics=("parallel", "arbitrary", "arbitrary")),
    )(dsq, msq, dvt, node0, wie, wbig, small)

    eqf = jnp.transpose(eqf_t, (0, 2, 1, 3))
    eqdr = jnp.transpose(eqdr_t, (0, 2, 1, 3))
    return node, eqF, eqf, eqdr
